# baseline (speedup 1.0000x reference)
import functools

import jax
import jax.numpy as jnp
from jax import lax
from jax.experimental import pallas as pl
from jax.experimental.pallas import tpu as pltpu
from jax.experimental.pallas import tpu_sc as plsc

NC, NS = 2, 16
NW = NC * NS
IDX_ROW = 128
CHUNK_IDX_ROWS = 4
NBUF = 2


def _gather(idx2d, weight):
    n_rows = idx2d.shape[0]
    D = weight.shape[1]
    rows_per_w = n_rows // NW
    CH = CHUNK_IDX_ROWS * IDX_ROW
    n_chunks = rows_per_w // CHUNK_IDX_ROWS
    B = n_rows * IDX_ROW

    mesh = plsc.VectorSubcoreMesh(
        core_axis_name="c", subcore_axis_name="s",
        num_cores=NC, num_subcores=NS)

    @functools.partial(
        pl.kernel,
        out_type=jax.ShapeDtypeStruct((B, D), jnp.float32),
        mesh=mesh,
        scratch_types=(
            [pltpu.VMEM((rows_per_w, IDX_ROW), jnp.int32)]
            + [pltpu.VMEM((CH, D), jnp.float32) for _ in range(NBUF)]
            + [pltpu.SemaphoreType.DMA for _ in range(2 * NBUF)]
        ),
        compiler_params=pltpu.CompilerParams(use_tc_tiling_on_sc=False),
    )
    def k(idx_hbm, table_hbm, out_hbm, idx_v, *bufs_and_sems):
        rows = bufs_and_sems[:NBUF]
        sg = bufs_and_sems[NBUF:2 * NBUF]
        so = bufs_and_sems[2 * NBUF:]
        wid = lax.axis_index("s") * NC + lax.axis_index("c")
        row0 = wid * rows_per_w
        base = row0 * IDX_ROW

        pltpu.sync_copy(idx_hbm.at[pl.ds(row0, rows_per_w)], idx_v)

        def fire_gather(g, b):
            for j in range(CHUNK_IDX_ROWS):
                pltpu.async_copy(
                    table_hbm.at[idx_v.at[g * CHUNK_IDX_ROWS + j]],
                    rows[b].at[pl.ds(j * IDX_ROW, IDX_ROW)],
                    sg[b])

        def wait_gather(b):
            for j in range(CHUNK_IDX_ROWS):
                pltpu.make_async_copy(
                    table_hbm.at[idx_v.at[0]],
                    rows[b].at[pl.ds(j * IDX_ROW, IDX_ROW)],
                    sg[b]).wait()

        def start_out(g, b):
            pltpu.async_copy(rows[b], out_hbm.at[pl.ds(base + g * CH, CH)], so[b])

        def wait_out(g, b):
            pltpu.make_async_copy(
                rows[b], out_hbm.at[pl.ds(base + g * CH, CH)], so[b]).wait()

        for h in range(NBUF - 1):
            fire_gather(h, h)

        def step(g, b):
            fb = (b + NBUF - 1) % NBUF

            @pl.when(g + NBUF - 1 < n_chunks)
            def _():
                @pl.when(g >= 1)
                def _():
                    wait_out(g - 1, fb)
                fire_gather(g + NBUF - 1, fb)

            wait_gather(b)
            start_out(g, b)

        @pl.loop(0, n_chunks, step=NBUF)
        def outer(t):
            for b in range(NBUF):
                step(t + b, b)

        for h in range(n_chunks - NBUF + 1, n_chunks):
            wait_out(h - 1, (h - 1) % NBUF)
        wait_out(n_chunks - 1, (n_chunks - 1) % NBUF)

    return k(idx2d, weight)


def kernel(input, weight):
    idx2d = input.reshape(-1, IDX_ROW).astype(jnp.int32)
    out = _gather(idx2d, weight)
    return out.reshape(input.shape + (weight.shape[1],))


# speedup vs baseline: 1.0008x; 1.0008x over previous
import functools

import jax
import jax.numpy as jnp
from jax import lax
from jax.experimental import pallas as pl
from jax.experimental.pallas import tpu as pltpu
from jax.experimental.pallas import tpu_sc as plsc

NC, NS = 2, 16
NW = NC * NS
R = 4
NBUF = 2


def _gather(idx, weight):
    n_in, T = idx.shape
    D = weight.shape[1]
    rows_per_w = n_in // NW
    n_chunks = rows_per_w // R
    SPLIT = 128

    mesh = plsc.VectorSubcoreMesh(
        core_axis_name="c", subcore_axis_name="s",
        num_cores=NC, num_subcores=NS)

    @functools.partial(
        pl.kernel,
        out_type=jax.ShapeDtypeStruct((n_in, T, D), jnp.float32),
        mesh=mesh,
        scratch_types=(
            [pltpu.VMEM((R, T), jnp.int32) for _ in range(NBUF)]
            + [pltpu.VMEM((R, T, D), jnp.float32) for _ in range(NBUF)]
            + [pltpu.SemaphoreType.DMA for _ in range(3 * NBUF)]
        ),
        compiler_params=pltpu.CompilerParams(use_tc_tiling_on_sc=False),
    )
    def k(idx_hbm, table_hbm, out_hbm, *refs):
        idx_v = refs[:NBUF]
        rows = refs[NBUF:2 * NBUF]
        si = refs[2 * NBUF:3 * NBUF]
        sg = refs[3 * NBUF:4 * NBUF]
        so = refs[4 * NBUF:]
        wid = lax.axis_index("s") * NC + lax.axis_index("c")
        row0 = wid * rows_per_w

        def idx_pair(g, b):
            return (idx_hbm.at[pl.ds(row0 + g * R, R)], idx_v[b], si[b])

        def out_pair(g, b):
            return (rows[b], out_hbm.at[pl.ds(row0 + g * R, R)], so[b])

        def fire_idx(g, b):
            pltpu.async_copy(*idx_pair(g, b))

        def wait_idx(g, b):
            pltpu.make_async_copy(*idx_pair(g, b)).wait()

        def fire_gather(b):
            for i in range(R):
                pltpu.async_copy(
                    table_hbm.at[idx_v[b].at[i, pl.ds(0, SPLIT)]],
                    rows[b].at[i, pl.ds(0, SPLIT)],
                    sg[b])
                pltpu.async_copy(
                    table_hbm.at[idx_v[b].at[i, pl.ds(SPLIT, T - SPLIT)]],
                    rows[b].at[i, pl.ds(SPLIT, T - SPLIT)],
                    sg[b])

        def wait_gather(b):
            for i in range(R):
                pltpu.make_async_copy(
                    table_hbm.at[idx_v[b].at[i, pl.ds(0, SPLIT)]],
                    rows[b].at[i, pl.ds(0, SPLIT)],
                    sg[b]).wait()
                pltpu.make_async_copy(
                    table_hbm.at[idx_v[b].at[i, pl.ds(SPLIT, T - SPLIT)]],
                    rows[b].at[i, pl.ds(SPLIT, T - SPLIT)],
                    sg[b]).wait()

        def start_out(g, b):
            pltpu.async_copy(*out_pair(g, b))

        def wait_out(g, b):
            pltpu.make_async_copy(*out_pair(g, b)).wait()

        fire_idx(0, 0)
        wait_idx(0, 0)
        fire_gather(0)
        fire_idx(1, 1)

        def step(g, b):
            nb = 1 - b

            @pl.when(g + 1 < n_chunks)
            def _():
                wait_idx(g + 1, nb)

                @pl.when(g >= 1)
                def _():
                    wait_out(g - 1, nb)

                fire_gather(nb)

            wait_gather(b)
            start_out(g, b)

            @pl.when(g + 2 < n_chunks)
            def _():
                fire_idx(g + 2, b)

        @pl.loop(0, n_chunks, step=NBUF)
        def outer(t):
            for b in range(NBUF):
                step(t + b, b)

        wait_out(n_chunks - 2, (n_chunks - 2) % NBUF)
        wait_out(n_chunks - 1, (n_chunks - 1) % NBUF)

    return k(idx, weight)


def kernel(input, weight):
    return _gather(input.astype(jnp.int32), weight)
